# baseline (speedup 1.0000x reference)
import jax
import jax.numpy as jnp
from jax.experimental import pallas as pl

_TQ = 256
_RC = 8
_L = 128


def _topk_interp_body(r1_ref, r2_ref, qp_ref, vtp_ref, feat_ref, out_ref):
    dotneg = jax.lax.dot_general(
        qp_ref[...], vtp_ref[...], (((1,), (0,)), ((), ())),
        preferred_element_type=jnp.float32)
    d = (r1_ref[...] + r2_ref[...]) + dotneg
    tq, kdim = d.shape
    ns = kdim // _L
    inf = jnp.float32(jnp.inf)
    fbig = jnp.float32(kdim)

    lane = jax.lax.broadcasted_iota(jnp.int32, (_RC, _L), 1).astype(jnp.float32)
    d1s, d2s, d3s, i1s, i2s, i3s = [], [], [], [], [], []
    for rb in range(tq // _RC):
        a1 = jnp.full((_RC, _L), inf)
        a2 = jnp.full((_RC, _L), inf)
        a3 = jnp.full((_RC, _L), inf)
        i1 = jnp.zeros((_RC, _L), jnp.float32)
        i2 = jnp.zeros((_RC, _L), jnp.float32)
        i3 = jnp.zeros((_RC, _L), jnp.float32)
        for s in range(ns):
            c = d[rb * _RC:(rb + 1) * _RC, s * _L:(s + 1) * _L]
            sid = jnp.float32(s)
            lt1 = c < a1
            lt2 = c < a2
            lt3 = c < a3
            a3n = jnp.where(lt3, jnp.where(lt2, a2, c), a3)
            i3n = jnp.where(lt3, jnp.where(lt2, i2, sid), i3)
            a2n = jnp.where(lt2, jnp.where(lt1, a1, c), a2)
            i2n = jnp.where(lt2, jnp.where(lt1, i1, sid), i2)
            a1 = jnp.where(lt1, c, a1)
            i1 = jnp.where(lt1, sid, i1)
            a2, a3, i2, i3 = a2n, a3n, i2n, i3n
        av = jnp.concatenate([a1, a2, a3], axis=1)
        gv = jnp.concatenate([i1 * _L + lane, i2 * _L + lane,
                              i3 * _L + lane], axis=1)
        for r in range(3):
            m = jnp.min(av, axis=1, keepdims=True)
            first = jnp.min(jnp.where(av == m, gv, fbig), axis=1, keepdims=True)
            (d1s, d2s, d3s)[r].append(m)
            (i1s, i2s, i3s)[r].append(first)
            if r < 2:
                av = jnp.where(gv == first, inf, av)
    d1 = jnp.concatenate(d1s, axis=0)
    d2 = jnp.concatenate(d2s, axis=0)
    d3 = jnp.concatenate(d3s, axis=0)
    j1 = jnp.concatenate(i1s, axis=0)
    j2 = jnp.concatenate(i2s, axis=0)
    j3 = jnp.concatenate(i3s, axis=0)
    e1 = jnp.ones_like(d1)
    e2 = jnp.exp(d1 - d2)
    e3 = jnp.exp(d1 - d3)
    s = e1 + e2 + e3
    fiota = jax.lax.broadcasted_iota(jnp.int32, d.shape, 1).astype(jnp.float32)
    w = (jnp.where(fiota == j1, e1 / s, 0.0)
         + jnp.where(fiota == j2, e2 / s, 0.0)
         + jnp.where(fiota == j3, e3 / s, 0.0))
    out_ref[...] = jax.lax.dot_general(
        w, feat_ref[...], (((1,), (0,)), ((), ())),
        preferred_element_type=jnp.float32)


def kernel(new_vertices, vertices, points_feat):
    q_total = new_vertices.shape[0]
    k_total, c_dim = points_feat.shape[1], points_feat.shape[2]
    r1 = jnp.sum(new_vertices ** 2, axis=-1)[:, None]
    r2 = jnp.sum(vertices ** 2, axis=-1)[None, :]
    qp = jnp.pad(-2.0 * new_vertices, ((0, 0), (0, 5))).astype(jnp.bfloat16)
    vtp = jnp.pad(vertices.T, ((0, 5), (0, 0))).astype(jnp.bfloat16)
    feat = points_feat[0]
    out = pl.pallas_call(
        _topk_interp_body,
        grid=(q_total // _TQ,),
        in_specs=[
            pl.BlockSpec((_TQ, 1), lambda i: (i, 0)),
            pl.BlockSpec((1, k_total), lambda i: (0, 0)),
            pl.BlockSpec((_TQ, 8), lambda i: (i, 0)),
            pl.BlockSpec((8, k_total), lambda i: (0, 0)),
            pl.BlockSpec((k_total, c_dim), lambda i: (0, 0)),
        ],
        out_specs=pl.BlockSpec((_TQ, c_dim), lambda i: (i, 0)),
        out_shape=jax.ShapeDtypeStruct((q_total, c_dim), jnp.float32),
    )(r1, r2, qp, vtp, feat)
    return out[None]


# speedup vs baseline: 1.0426x; 1.0426x over previous
import functools

import jax
import jax.numpy as jnp
from jax import lax
from jax.experimental import pallas as pl
from jax.experimental.pallas import tpu as pltpu
from jax.experimental.pallas import tpu_sc as plsc

_TQ = 256
_QC = 128
_NW = 32


def _topk_body(r1_ref, r2_ref, qp_ref, vtp_ref,
               j1_ref, j2_ref, j3_ref, w1_ref, w2_ref, w3_ref):
    dotneg = jax.lax.dot_general(
        qp_ref[...], vtp_ref[...], (((1,), (0,)), ((), ())),
        preferred_element_type=jnp.float32)
    d = (r1_ref[...] + r2_ref[...]) + dotneg
    kdim = d.shape[1]
    fiota = jax.lax.broadcasted_iota(jnp.int32, d.shape, 1).astype(jnp.float32)
    fbig = jnp.float32(kdim)
    mins, idxs = [], []
    dd = d
    for r in range(3):
        m = jnp.min(dd, axis=1, keepdims=True)
        first = jnp.min(jnp.where(dd == m, fiota, fbig), axis=1, keepdims=True)
        mins.append(m)
        idxs.append(first)
        if r < 2:
            dd = jnp.where(fiota == first, jnp.float32(jnp.inf), dd)
    d1, d2, d3 = mins
    e2 = jnp.exp(d1 - d2)
    e3 = jnp.exp(d1 - d3)
    s = 1.0 + e2 + e3
    j1_ref[...] = idxs[0].astype(jnp.int32)
    j2_ref[...] = idxs[1].astype(jnp.int32)
    j3_ref[...] = idxs[2].astype(jnp.int32)
    w1_ref[...] = 1.0 / s
    w2_ref[...] = e2 / s
    w3_ref[...] = e3 / s


def _tc_topk(new_vertices, vertices, q_total, k_total):
    r1 = jnp.sum(new_vertices ** 2, axis=-1)[:, None]
    r2 = jnp.sum(vertices ** 2, axis=-1)[None, :]
    qp = jnp.pad(-2.0 * new_vertices, ((0, 0), (0, 5))).astype(jnp.bfloat16)
    vtp = jnp.pad(vertices.T, ((0, 5), (0, 0))).astype(jnp.bfloat16)
    col = pl.BlockSpec((_TQ, 1), lambda i: (i, 0))
    return pl.pallas_call(
        _topk_body,
        grid=(q_total // _TQ,),
        in_specs=[
            col,
            pl.BlockSpec((1, k_total), lambda i: (0, 0)),
            pl.BlockSpec((_TQ, 8), lambda i: (i, 0)),
            pl.BlockSpec((8, k_total), lambda i: (0, 0)),
        ],
        out_specs=[col] * 6,
        out_shape=[jax.ShapeDtypeStruct((q_total, 1), jnp.int32)] * 3
        + [jax.ShapeDtypeStruct((q_total, 1), jnp.float32)] * 3,
    )(r1, r2, qp, vtp)


def _sc_interp_body(feat_hbm, i1_hbm, i2_hbm, i3_hbm, w1_hbm, w2_hbm, w3_hbm,
                    out_hbm, i1_v, i2_v, i3_v, w1_v, w2_v, w3_v,
                    r1_v, r2_v, r3_v, out_v, sem1, sem2, sem3):
    q_total = out_hbm.shape[0]
    qpw = q_total // _NW
    wid = lax.axis_index("s") * 2 + lax.axis_index("c")
    base = wid * qpw

    def chunk_body(ci, carry):
        qb = base + ci * _QC
        pltpu.sync_copy(i1_hbm.at[pl.ds(qb, _QC)], i1_v)
        pltpu.sync_copy(i2_hbm.at[pl.ds(qb, _QC)], i2_v)
        pltpu.sync_copy(i3_hbm.at[pl.ds(qb, _QC)], i3_v)
        pltpu.sync_copy(w1_hbm.at[pl.ds(qb, _QC)], w1_v)
        pltpu.sync_copy(w2_hbm.at[pl.ds(qb, _QC)], w2_v)
        pltpu.sync_copy(w3_hbm.at[pl.ds(qb, _QC)], w3_v)
        cp1 = pltpu.async_copy(feat_hbm.at[i1_v], r1_v, sem1)
        cp2 = pltpu.async_copy(feat_hbm.at[i2_v], r2_v, sem2)
        cp3 = pltpu.async_copy(feat_hbm.at[i3_v], r3_v, sem3)
        cp1.wait()
        cp2.wait()
        cp3.wait()

        def group_body(g, c2):
            wv1 = w1_v[pl.ds(g * 16, 16)]
            wv2 = w2_v[pl.ds(g * 16, 16)]
            wv3 = w3_v[pl.ds(g * 16, 16)]
            for i in range(16):
                q = g * 16 + i
                a, b, c = wv1[i], wv2[i], wv3[i]
                for cb in range(8):
                    sl = pl.ds(cb * 16, 16)
                    out_v[q, sl] = (a * r1_v[q, sl] + b * r2_v[q, sl]
                                    + c * r3_v[q, sl])
            return c2

        lax.fori_loop(0, _QC // 16, group_body, 0, unroll=False)
        pltpu.sync_copy(out_v, out_hbm.at[pl.ds(qb, _QC)])
        return carry

    lax.fori_loop(0, qpw // _QC, chunk_body, 0, unroll=False)


def _sc_interp(feat, i1, i2, i3, w1, w2, w3, q_total, c_dim):
    mesh = plsc.VectorSubcoreMesh(core_axis_name="c", subcore_axis_name="s")
    fn = pl.kernel(
        _sc_interp_body,
        mesh=mesh,
        out_type=jax.ShapeDtypeStruct((q_total, c_dim), jnp.float32),
        scratch_types=[
            pltpu.VMEM((_QC,), jnp.int32),
            pltpu.VMEM((_QC,), jnp.int32),
            pltpu.VMEM((_QC,), jnp.int32),
            pltpu.VMEM((_QC,), jnp.float32),
            pltpu.VMEM((_QC,), jnp.float32),
            pltpu.VMEM((_QC,), jnp.float32),
            pltpu.VMEM((_QC, c_dim), jnp.float32),
            pltpu.VMEM((_QC, c_dim), jnp.float32),
            pltpu.VMEM((_QC, c_dim), jnp.float32),
            pltpu.VMEM((_QC, c_dim), jnp.float32),
            pltpu.SemaphoreType.DMA,
            pltpu.SemaphoreType.DMA,
            pltpu.SemaphoreType.DMA,
        ],
    )
    return fn(feat, i1, i2, i3, w1, w2, w3)


def kernel(new_vertices, vertices, points_feat):
    q_total = new_vertices.shape[0]
    k_total, c_dim = points_feat.shape[1], points_feat.shape[2]
    j1, j2, j3, w1, w2, w3 = _tc_topk(new_vertices, vertices, q_total, k_total)
    feat = points_feat[0]
    out = _sc_interp(feat,
                     j1.reshape(q_total), j2.reshape(q_total),
                     j3.reshape(q_total),
                     w1.reshape(q_total), w2.reshape(q_total),
                     w3.reshape(q_total), q_total, c_dim)
    return out[None]


# speedup vs baseline: 1.0592x; 1.0159x over previous
import functools

import jax
import jax.numpy as jnp
from jax import lax
from jax.experimental import pallas as pl
from jax.experimental.pallas import tpu as pltpu
from jax.experimental.pallas import tpu_sc as plsc

_TQ = 256
_QC = 128
_NW = 32


def _topk_body(r1_ref, r2_ref, qp_ref, vtp_ref,
               j1_ref, j2_ref, j3_ref, w1_ref, w2_ref, w3_ref):
    dotneg = jax.lax.dot_general(
        qp_ref[...], vtp_ref[...], (((1,), (0,)), ((), ())),
        preferred_element_type=jnp.float32)
    d = (r1_ref[...] + r2_ref[...]) + dotneg
    kdim = d.shape[1]
    fiota = jax.lax.broadcasted_iota(jnp.int32, d.shape, 1).astype(jnp.float32)
    fbig = jnp.float32(kdim)
    mins, idxs = [], []
    dd = d
    for r in range(3):
        m = jnp.min(dd, axis=1, keepdims=True)
        first = jnp.min(jnp.where(dd == m, fiota, fbig), axis=1, keepdims=True)
        mins.append(m)
        idxs.append(first)
        if r < 2:
            dd = jnp.where(fiota == first, jnp.float32(jnp.inf), dd)
    d1, d2, d3 = mins
    e2 = jnp.exp(d1 - d2)
    e3 = jnp.exp(d1 - d3)
    s = 1.0 + e2 + e3
    j1_ref[...] = idxs[0].astype(jnp.int32)
    j2_ref[...] = idxs[1].astype(jnp.int32)
    j3_ref[...] = idxs[2].astype(jnp.int32)
    w1_ref[...] = 1.0 / s
    w2_ref[...] = e2 / s
    w3_ref[...] = e3 / s


def _tc_topk(new_vertices, vertices, q_total, k_total):
    r1 = jnp.sum(new_vertices ** 2, axis=-1)[:, None]
    r2 = jnp.sum(vertices ** 2, axis=-1)[None, :]
    qp = jnp.pad(-2.0 * new_vertices, ((0, 0), (0, 5))).astype(jnp.bfloat16)
    vtp = jnp.pad(vertices.T, ((0, 5), (0, 0))).astype(jnp.bfloat16)
    col = pl.BlockSpec((_TQ, 1), lambda i: (i, 0))
    return pl.pallas_call(
        _topk_body,
        grid=(q_total // _TQ,),
        in_specs=[
            col,
            pl.BlockSpec((1, k_total), lambda i: (0, 0)),
            pl.BlockSpec((_TQ, 8), lambda i: (i, 0)),
            pl.BlockSpec((8, k_total), lambda i: (0, 0)),
        ],
        out_specs=[col] * 6,
        out_shape=[jax.ShapeDtypeStruct((q_total, 1), jnp.int32)] * 3
        + [jax.ShapeDtypeStruct((q_total, 1), jnp.float32)] * 3,
    )(r1, r2, qp, vtp)


def _sc_interp_body(feat_hbm, i1_hbm, i2_hbm, i3_hbm, w1_hbm, w2_hbm, w3_hbm,
                    out_hbm, i1_v, i2_v, i3_v, w1_v, w2_v, w3_v,
                    r1_v, r2_v, r3_v, out_v, sem1, sem2, sem3):
    q_total = out_hbm.shape[0]
    qpw = q_total // _NW
    wid = lax.axis_index("s") * 2 + lax.axis_index("c")
    base = wid * qpw

    def chunk_body(ci, carry):
        qb = base + ci * _QC
        pltpu.sync_copy(i1_hbm.at[pl.ds(qb, _QC)], i1_v)
        pltpu.sync_copy(i2_hbm.at[pl.ds(qb, _QC)], i2_v)
        pltpu.sync_copy(i3_hbm.at[pl.ds(qb, _QC)], i3_v)
        pltpu.sync_copy(w1_hbm.at[pl.ds(qb, _QC)], w1_v)
        pltpu.sync_copy(w2_hbm.at[pl.ds(qb, _QC)], w2_v)
        pltpu.sync_copy(w3_hbm.at[pl.ds(qb, _QC)], w3_v)
        cp1 = pltpu.async_copy(feat_hbm.at[i1_v], r1_v, sem1)
        cp2 = pltpu.async_copy(feat_hbm.at[i2_v], r2_v, sem2)
        cp3 = pltpu.async_copy(feat_hbm.at[i3_v], r3_v, sem3)
        cp1.wait()
        cp2.wait()
        cp3.wait()

        def group_body(g, c2):
            wv1 = w1_v[pl.ds(g * 16, 16)]
            wv2 = w2_v[pl.ds(g * 16, 16)]
            wv3 = w3_v[pl.ds(g * 16, 16)]
            for i in range(16):
                q = g * 16 + i
                a, b, c = wv1[i], wv2[i], wv3[i]
                for cb in range(8):
                    sl = pl.ds(cb * 16, 16)
                    out_v[q, sl] = (a * r1_v[q, sl] + b * r2_v[q, sl]
                                    + c * r3_v[q, sl])
            return c2

        lax.fori_loop(0, _QC // 16, group_body, 0, unroll=False)
        pltpu.sync_copy(out_v, out_hbm.at[pl.ds(qb, _QC)])
        return carry

    lax.fori_loop(0, qpw // _QC, chunk_body, 0, unroll=False)


def _sc_interp(feat, i1, i2, i3, w1, w2, w3, q_total, c_dim):
    mesh = plsc.VectorSubcoreMesh(core_axis_name="c", subcore_axis_name="s")
    fn = pl.kernel(
        _sc_interp_body,
        mesh=mesh,
        out_type=jax.ShapeDtypeStruct((q_total, c_dim), jnp.float32),
        scratch_types=[
            pltpu.VMEM((_QC,), jnp.int32),
            pltpu.VMEM((_QC,), jnp.int32),
            pltpu.VMEM((_QC,), jnp.int32),
            pltpu.VMEM((_QC,), jnp.float32),
            pltpu.VMEM((_QC,), jnp.float32),
            pltpu.VMEM((_QC,), jnp.float32),
            pltpu.VMEM((_QC, c_dim), jnp.float32),
            pltpu.VMEM((_QC, c_dim), jnp.float32),
            pltpu.VMEM((_QC, c_dim), jnp.float32),
            pltpu.VMEM((_QC, c_dim), jnp.float32),
            pltpu.SemaphoreType.DMA,
            pltpu.SemaphoreType.DMA,
            pltpu.SemaphoreType.DMA,
        ],
    )
    return fn(feat, i1, i2, i3, w1, w2, w3)


def kernel(new_vertices, vertices, points_feat):
    q_total = new_vertices.shape[0]
    k_total, c_dim = points_feat.shape[1], points_feat.shape[2]
    feat = points_feat[0]
    nseg = 2
    qs = q_total // nseg
    outs = []
    for s in range(nseg):
        qv = jax.lax.slice_in_dim(new_vertices, s * qs, (s + 1) * qs, axis=0)
        j1, j2, j3, w1, w2, w3 = _tc_topk(qv, vertices, qs, k_total)
        outs.append(_sc_interp(feat,
                               j1.reshape(qs), j2.reshape(qs), j3.reshape(qs),
                               w1.reshape(qs), w2.reshape(qs), w3.reshape(qs),
                               qs, c_dim))
    return jnp.concatenate(outs, axis=0)[None]
